# Initial kernel scaffold; baseline (speedup 1.0000x reference)
import jax
import jax.numpy as jnp
from jax.experimental import pallas as pl


def kernel(sequences, edge_index1, edge_index2, node_table, Wg1, bg1, Wg2, bg2, Wq, bq, Wk, bk, Wv, bv, Wo, bo, ln1_g, ln1_b, ln2_g, ln2_b, W1, b1, W2, b2):
    raise NotImplementedError("write your pallas kernel here")


# speedup vs baseline: 2.8103x; 2.8103x over previous
import functools
from typing import Any

import jax
import jax.numpy as jnp
import numpy as np
from jax import lax
from jax.experimental import pallas as pl
from jax.experimental.pallas import tpu as pltpu
from jax.experimental.pallas import tpu_sc as plsc

NH = 4
S_PAD = 64
NQ = 4
QW = 32
CH = 128


def _sc_segsum(table_q, e1, e2, VP):
    E_pad = e1.shape[1]
    per_tile = E_pad // 16
    n_chunks = per_tile // CH
    stripe = VP // 16
    ZR = 448
    assert stripe % ZR == 0 and ZR % 16 == 0
    mesh = plsc.VectorSubcoreMesh(core_axis_name="c", subcore_axis_name="s")

    @functools.partial(
        pl.kernel, mesh=mesh,
        compiler_params=pltpu.CompilerParams(use_tc_tiling_on_sc=False),
        out_type=[jax.ShapeDtypeStruct((NQ, VP, QW), jnp.float32),
                  jax.ShapeDtypeStruct((NQ, VP, QW), jnp.float32),
                  jax.ShapeDtypeStruct((VP,), jnp.float32),
                  jax.ShapeDtypeStruct((VP,), jnp.float32)],
        scratch_types=[pltpu.VMEM((CH,), jnp.int32),
                       pltpu.VMEM((CH,), jnp.int32),
                       pltpu.VMEM((CH, QW), jnp.float32),
                       pltpu.VMEM((CH,), jnp.float32),
                       pltpu.VMEM((ZR, QW), jnp.float32),
                       pltpu.VMEM((ZR,), jnp.float32),
                       pltpu.VMEM_SHARED((VP, QW), jnp.float32),
                       pltpu.VMEM_SHARED((VP,), jnp.float32)],
    )
    def k(tq_hbm, e1_hbm, e2_hbm, agg1_hbm, agg2_hbm, deg1_hbm, deg2_hbm,
          src_v, dst_v, rows_v, ones_v, zrow_v, zdeg_v, sc_shared, deg_shared):
        cid = lax.axis_index("c")
        tid = lax.axis_index("s")

        def fill2d(ref, n, val):
            def b(i, _):
                ref[i, pl.ds(0, 16)] = jnp.full((16,), val, jnp.float32)
                ref[i, pl.ds(16, 16)] = jnp.full((16,), val, jnp.float32)
                return ()
            lax.fori_loop(0, n, b, ())

        def fill1d(ref, n, val):
            def b(i, _):
                ref[pl.ds(i * 16, 16)] = jnp.full((16,), val, jnp.float32)
                return ()
            lax.fori_loop(0, n // 16, b, ())

        fill2d(zrow_v, ZR, 0.0)
        fill1d(zdeg_v, ZR, 0.0)
        fill1d(ones_v, CH, 1.0)

        def run_set(e_hbm, agg_hbm, deg_hbm, sc_shared, deg_shared):
            base = tid * per_tile
            for q in range(NQ):
                def zchunk(j, _):
                    pltpu.sync_copy(
                        zrow_v, sc_shared.at[pl.ds(tid * stripe + j * ZR, ZR)])
                    if q == 0:
                        pltpu.sync_copy(
                            zdeg_v,
                            deg_shared.at[pl.ds(tid * stripe + j * ZR, ZR)])
                    return ()
                lax.fori_loop(0, stripe // ZR, zchunk, ())
                plsc.subcore_barrier()

                def chunk(i, _):
                    off = base + i * CH
                    pltpu.sync_copy(e_hbm.at[0, pl.ds(off, CH)], src_v)
                    pltpu.sync_copy(e_hbm.at[1, pl.ds(off, CH)], dst_v)
                    pltpu.sync_copy(tq_hbm.at[q].at[src_v], rows_v)
                    pltpu.sync_copy(rows_v, sc_shared.at[dst_v], add=True)
                    if q == 0:
                        pltpu.sync_copy(ones_v, deg_shared.at[dst_v], add=True)
                    return ()
                lax.fori_loop(0, n_chunks, chunk, ())
                plsc.subcore_barrier()

                sl = pl.ds(tid * stripe, stripe)
                pltpu.sync_copy(sc_shared.at[sl], agg_hbm.at[q].at[sl])
                if q == 0:
                    pltpu.sync_copy(deg_shared.at[sl], deg_hbm.at[sl])
                plsc.subcore_barrier()

        @pl.when(cid == 0)
        def _():
            run_set(e1_hbm, agg1_hbm, deg1_hbm, sc_shared, deg_shared)

        @pl.when(cid == 1)
        def _():
            run_set(e2_hbm, agg2_hbm, deg2_hbm, sc_shared, deg_shared)

    return k


def _sc_lookup(enc1, enc2, idx2d):
    NR, W = idx2d.shape
    V, H = enc1.shape
    rows_per_w = NR // 32
    mesh = plsc.VectorSubcoreMesh(core_axis_name="c", subcore_axis_name="s")

    @functools.partial(
        pl.kernel, mesh=mesh,
        out_type=jax.ShapeDtypeStruct((2 * NR * W, H), jnp.float32),
        scratch_types=[pltpu.VMEM((W,), jnp.int32),
                       pltpu.VMEM((W, H), jnp.float32),
                       pltpu.VMEM((W, H), jnp.float32),
                       pltpu.SemaphoreType.DMA,
                       pltpu.SemaphoreType.DMA],
    )
    def k(enc1_hbm, enc2_hbm, idx_hbm, out_hbm, idx_v, r1_v, r2_v, sem1, sem2):
        wid = lax.axis_index("s") * 2 + lax.axis_index("c")

        def body(r, _):
            row = wid * rows_per_w + r
            pltpu.sync_copy(idx_hbm.at[row], idx_v)
            cp1 = pltpu.async_copy(enc1_hbm.at[idx_v], r1_v, sem1)
            cp2 = pltpu.async_copy(enc2_hbm.at[idx_v], r2_v, sem2)
            cp1.wait()
            pltpu.sync_copy(r1_v, out_hbm.at[pl.ds(row * W, W)])
            cp2.wait()
            pltpu.sync_copy(r2_v, out_hbm.at[pl.ds((NR + row) * W, W)])
            return ()
        lax.fori_loop(0, rows_per_w, body, ())

    return k(enc1, enc2, idx2d)


def _gcn_finish_q_body(agg1, deg1, agg2, deg2, wg1, bg1, wg2, bg2, out1, out2):
    d1 = jnp.maximum(deg1[...], 1.0)
    d2 = jnp.maximum(deg2[...], 1.0)
    nq = agg1.shape[0]
    qw = agg1.shape[2]
    acc1 = bg1[...] * 1.0
    acc2 = bg2[...] * 1.0
    for q in range(nq):
        acc1 = acc1 + jnp.dot(agg1[q] / d1, wg1[pl.ds(q * qw, qw), :],
                              preferred_element_type=jnp.float32)
        acc2 = acc2 + jnp.dot(agg2[q] / d2, wg2[pl.ds(q * qw, qw), :],
                              preferred_element_type=jnp.float32)
    out1[...] = jnp.maximum(acc1, 0.0)
    out2[...] = jnp.maximum(acc2, 0.0)


def _gcn_finish_q(agg1q, deg1, agg2q, deg2, Wg1, bg1, Wg2, bg2, V, vb):
    nq, VP, qw = agg1q.shape
    D, H = Wg1.shape
    grid = (V // vb,)
    bs_a = pl.BlockSpec((nq, vb, qw), lambda i: (0, i, 0))
    bs_d = pl.BlockSpec((vb, 1), lambda i: (i, 0))
    bs_w = pl.BlockSpec((D, H), lambda i: (0, 0))
    bs_b = pl.BlockSpec((1, H), lambda i: (0, 0))
    return pl.pallas_call(
        _gcn_finish_q_body,
        grid=grid,
        in_specs=[bs_a, bs_d, bs_a, bs_d, bs_w, bs_b, bs_w, bs_b],
        out_specs=[pl.BlockSpec((vb, H), lambda i: (i, 0))] * 2,
        out_shape=[jax.ShapeDtypeStruct((V, H), jnp.float32)] * 2,
    )(agg1q, deg1[:, None], agg2q, deg2[:, None], Wg1, bg1[None], Wg2, bg2[None])


def _encoder_body(x_ref, padf_ref, wq, bq, wk, bk, wv, bv, wo, bo,
                  ln1g, ln1b, ln2g, ln2b, w1, b1, w2, b2, pooled_ref, *, bb, dh):
    sp = x_ref.shape[1]
    h = x_ref.shape[2]
    x = x_ref[...]
    x2 = x.reshape(bb * sp, h)
    padf = padf_ref[...]
    neg = padf * -1e9

    o_acc = jnp.zeros((bb * sp, h), jnp.float32)
    scale = 1.0 / np.sqrt(dh)
    nh = h // dh
    for hd in range(nh):
        wq_h = wq[:, hd * dh:(hd + 1) * dh]
        wk_h = wk[:, hd * dh:(hd + 1) * dh]
        wv_h = wv[:, hd * dh:(hd + 1) * dh]
        bq_h = bq[:, hd * dh:(hd + 1) * dh]
        bk_h = bk[:, hd * dh:(hd + 1) * dh]
        bv_h = bv[:, hd * dh:(hd + 1) * dh]
        q_h = (jnp.dot(x2, wq_h, preferred_element_type=jnp.float32) + bq_h
               ).reshape(bb, sp, dh)
        k_h = (jnp.dot(x2, wk_h, preferred_element_type=jnp.float32) + bk_h
               ).reshape(bb, sp, dh)
        v_h = (jnp.dot(x2, wv_h, preferred_element_type=jnp.float32) + bv_h
               ).reshape(bb, sp, dh)
        scores = jax.lax.dot_general(
            q_h, k_h, (((2,), (2,)), ((0,), (0,))),
            preferred_element_type=jnp.float32) * scale
        scores = scores + neg[:, None, :]
        scores = scores - jnp.max(scores, axis=-1, keepdims=True)
        e = jnp.exp(scores)
        attn = e / jnp.sum(e, axis=-1, keepdims=True)
        o_h = jax.lax.dot_general(
            attn, v_h, (((2,), (1,)), ((0,), (0,))),
            preferred_element_type=jnp.float32)
        wo_h = wo[hd * dh:(hd + 1) * dh, :]
        o_acc = o_acc + jnp.dot(o_h.reshape(bb * sp, dh), wo_h,
                                preferred_element_type=jnp.float32)
    o_acc = o_acc + bo[...]

    def ln(t, g, b):
        mu = jnp.mean(t, axis=-1, keepdims=True)
        var = jnp.mean((t - mu) ** 2, axis=-1, keepdims=True)
        return (t - mu) / jnp.sqrt(var + 1e-5) * g[...] + b[...]

    x2 = ln(x2 + o_acc, ln1g, ln1b)
    f = jnp.maximum(jnp.dot(x2, w1[...], preferred_element_type=jnp.float32)
                    + b1[...], 0.0)
    f = jnp.dot(f, w2[...], preferred_element_type=jnp.float32) + b2[...]
    x2 = ln(x2 + f, ln2g, ln2b)

    enc = x2.reshape(bb, sp, h)
    keep = (1.0 - padf)
    summed = jnp.sum(enc * keep[:, :, None], axis=1)
    cnt = jnp.sum(keep, axis=1, keepdims=True)
    pooled_ref[...] = summed / cnt


def _encoder_pool(emb_all, padf, p, bb):
    twob, sp, h = emb_all.shape
    b = twob // 2
    dh = h // NH
    nblk = twob // bb
    bpb = b // bb
    bs_x = pl.BlockSpec((bb, sp, h), lambda i: (i, 0, 0))
    bs_m = pl.BlockSpec((bb, sp), lambda i: (i % bpb, 0))
    full = lambda *shape: pl.BlockSpec(shape, lambda i: (0,) * len(shape))
    w = lambda a: full(*a.shape)
    body = functools.partial(_encoder_body, bb=bb, dh=dh)
    return pl.pallas_call(
        body,
        grid=(nblk,),
        in_specs=[bs_x, bs_m,
                  w(p['Wq']), full(1, h), w(p['Wk']), full(1, h),
                  w(p['Wv']), full(1, h), w(p['Wo']), full(1, h),
                  full(1, h), full(1, h), full(1, h), full(1, h),
                  w(p['W1']), full(1, p['W1'].shape[1]),
                  w(p['W2']), full(1, h)],
        out_specs=pl.BlockSpec((bb, h), lambda i: (i, 0)),
        out_shape=jax.ShapeDtypeStruct((twob, h), jnp.float32),
    )(emb_all, padf,
      p['Wq'], p['bq'][None], p['Wk'], p['bk'][None],
      p['Wv'], p['bv'][None], p['Wo'], p['bo'][None],
      p['ln1_g'][None], p['ln1_b'][None], p['ln2_g'][None], p['ln2_b'][None],
      p['W1'], p['b1'][None], p['W2'], p['b2'][None])


def kernel(sequences, edge_index1, edge_index2, node_table, Wg1, bg1, Wg2, bg2,
           Wq, bq, Wk, bk, Wv, bv, Wo, bo, ln1_g, ln1_b, ln2_g, ln2_b,
           W1, b1, W2, b2):
    V, D = node_table.shape
    B, S = sequences.shape
    H = Wg1.shape[1]
    E = edge_index1.shape[1]

    grp = 16 * CH
    E_pad = ((E + grp - 1) // grp) * grp
    VP = ((V + 255) // 256) * 256
    npad = E_pad - E
    pad_src = (jnp.arange(npad, dtype=jnp.int32) % V)
    pad_dst = V + (jnp.arange(npad, dtype=jnp.int32) % (VP - V))
    pad_e = jnp.stack([pad_src, pad_dst])
    e1 = jnp.concatenate([edge_index1, pad_e], axis=1)
    e2 = jnp.concatenate([edge_index2, pad_e], axis=1)
    table_q = node_table.reshape(V, NQ, QW).transpose(1, 0, 2)

    seg_k = _sc_segsum(table_q, e1, e2, VP)
    agg1q, agg2q, deg1, deg2 = seg_k(table_q, e1, e2)

    vb = 2000 if V % 2000 == 0 else V
    node_enc1, node_enc2 = _gcn_finish_q(agg1q, deg1[:V], agg2q, deg2[:V],
                                         Wg1, bg1, Wg2, bg2, V, vb)

    sp = S_PAD if S <= S_PAD else S
    seq_pad = jnp.full((B, sp), V, jnp.int32).at[:, :S].set(sequences)
    padf = (seq_pad == V).astype(jnp.float32)
    flat = seq_pad.reshape(-1)
    fill = jnp.arange(flat.shape[0], dtype=jnp.int32) % V
    idx_eff = jnp.where(flat == V, fill, flat)
    idx2d = idx_eff.reshape(-1, 128)
    emb_flat = _sc_lookup(node_enc1, node_enc2, idx2d)
    emb_all = emb_flat.reshape(2 * B, sp, H)

    p = dict(Wq=Wq, bq=bq, Wk=Wk, bk=bk, Wv=Wv, bv=bv, Wo=Wo, bo=bo,
             ln1_g=ln1_g, ln1_b=ln1_b, ln2_g=ln2_g, ln2_b=ln2_b,
             W1=W1, b1=b1, W2=W2, b2=b2)
    bb = 64 if B % 64 == 0 else B
    pooled_all = _encoder_pool(emb_all, padf, p, bb)
    pooled1, pooled2 = pooled_all[:B], pooled_all[B:]
    return (node_enc1, node_enc2, pooled1, pooled2)


# speedup vs baseline: 4.7770x; 1.6998x over previous
import functools
from typing import Any

import jax
import jax.numpy as jnp
import numpy as np
from jax import lax
from jax.experimental import pallas as pl
from jax.experimental.pallas import tpu as pltpu
from jax.experimental.pallas import tpu_sc as plsc

NH = 4
S_PAD = 64
NQ = 4
QW = 32
CH = 128


def _sc_segsum(table_q, e1, e2, VP):
    NC = e1.shape[0]
    n_chunks = NC // 16
    per_tile = n_chunks
    stripe = VP // 16
    ZR = 224
    assert stripe % ZR == 0 and ZR % 16 == 0 and n_chunks % 4 == 0
    mesh = plsc.VectorSubcoreMesh(core_axis_name="c", subcore_axis_name="s")

    @functools.partial(
        pl.kernel, mesh=mesh,
        compiler_params=pltpu.CompilerParams(use_tc_tiling_on_sc=False),
        out_type=[jax.ShapeDtypeStruct((NQ, VP, QW), jnp.float32),
                  jax.ShapeDtypeStruct((NQ, VP, QW), jnp.float32),
                  jax.ShapeDtypeStruct((VP,), jnp.float32),
                  jax.ShapeDtypeStruct((VP,), jnp.float32)],
        scratch_types=[[pltpu.VMEM((2, CH), jnp.int32) for _ in range(4)],
                       [pltpu.VMEM((CH, QW), jnp.float32) for _ in range(2)],
                       pltpu.VMEM((CH,), jnp.float32),
                       pltpu.VMEM((ZR, QW), jnp.float32),
                       pltpu.VMEM((ZR,), jnp.float32),
                       [pltpu.SemaphoreType.DMA for _ in range(4)],
                       [pltpu.SemaphoreType.DMA for _ in range(2)],
                       [pltpu.SemaphoreType.DMA for _ in range(2)],
                       pltpu.SemaphoreType.DMA,
                       pltpu.VMEM_SHARED((VP, QW), jnp.float32),
                       pltpu.VMEM_SHARED((VP,), jnp.float32)],
    )
    def k(tq_hbm, e1_hbm, e2_hbm, agg1_hbm, agg2_hbm, deg1_hbm, deg2_hbm,
          ed_v, rows_v, ones_v, zrow_v, zdeg_v,
          sem_e, sem_g, sem_s, sem_deg, sc_shared, deg_shared):
        cid = lax.axis_index("c")
        tid = lax.axis_index("s")

        def fill2d(ref, n, val):
            def b(i, _):
                ref[i, pl.ds(0, 16)] = jnp.full((16,), val, jnp.float32)
                ref[i, pl.ds(16, 16)] = jnp.full((16,), val, jnp.float32)
                return ()
            lax.fori_loop(0, n, b, ())

        def fill1d(ref, n, val):
            def b(i, _):
                ref[pl.ds(i * 16, 16)] = jnp.full((16,), val, jnp.float32)
                return ()
            lax.fori_loop(0, n // 16, b, ())

        fill2d(zrow_v, ZR, 0.0)
        fill1d(zdeg_v, ZR, 0.0)
        fill1d(ones_v, CH, 1.0)

        def run_set(e_hbm, agg_hbm, deg_hbm):
            base = tid * per_tile
            n = n_chunks

            def wait_g(b):
                pltpu.make_async_copy(
                    tq_hbm.at[0].at[pl.ds(0, CH)], rows_v[b], sem_g[b]).wait()

            def wait_s(b):
                pltpu.make_async_copy(
                    rows_v[b], sc_shared.at[pl.ds(0, CH)], sem_s[b]).wait()

            for q in range(NQ):
                tq = tq_hbm.at[q]

                def zchunk(j, _):
                    pltpu.sync_copy(
                        zrow_v, sc_shared.at[pl.ds(tid * stripe + j * ZR, ZR)])
                    if q == 0:
                        pltpu.sync_copy(
                            zdeg_v,
                            deg_shared.at[pl.ds(tid * stripe + j * ZR, ZR)])
                    return ()
                lax.fori_loop(0, stripe // ZR, zchunk, ())
                plsc.subcore_barrier()

                pltpu.async_copy(e_hbm.at[base], ed_v[0], sem_e[0])
                pltpu.async_copy(e_hbm.at[base + 1], ed_v[1], sem_e[1])
                pltpu.make_async_copy(e_hbm.at[base], ed_v[0], sem_e[0]).wait()
                pltpu.async_copy(tq.at[ed_v[0].at[0]], rows_v[0], sem_g[0])

                def group(g4, _):
                    for j in range(4):
                        g = g4 * 4 + j
                        br, be = j % 2, j
                        @pl.when(g >= 1)
                        def _():
                            wait_s((j + 1) % 2)
                        @pl.when(g + 2 < n)
                        def _():
                            pltpu.async_copy(e_hbm.at[base + g + 2],
                                             ed_v[(j + 2) % 4], sem_e[(j + 2) % 4])
                        wait_g(br)
                        pltpu.async_copy(rows_v[br],
                                         sc_shared.at[ed_v[be].at[1]],
                                         sem_s[br], add=True)
                        if q == 0:
                            pltpu.async_copy(ones_v,
                                             deg_shared.at[ed_v[be].at[1]],
                                             sem_deg, add=True)
                        @pl.when(g + 1 < n)
                        def _():
                            pltpu.make_async_copy(
                                e_hbm.at[base], ed_v[(j + 1) % 4],
                                sem_e[(j + 1) % 4]).wait()
                            pltpu.async_copy(tq.at[ed_v[(j + 1) % 4].at[0]],
                                             rows_v[(j + 1) % 2],
                                             sem_g[(j + 1) % 2])
                    return ()
                lax.fori_loop(0, n // 4, group, ())

                wait_s((n - 1) % 2)
                if q == 0:
                    def dr(i, _):
                        pltpu.make_async_copy(
                            ones_v, deg_shared.at[pl.ds(0, CH)], sem_deg).wait()
                        return ()
                    lax.fori_loop(0, n, dr, ())
                plsc.subcore_barrier()

                sl = pl.ds(tid * stripe, stripe)
                pltpu.sync_copy(sc_shared.at[sl], agg_hbm.at[q].at[sl])
                if q == 0:
                    pltpu.sync_copy(deg_shared.at[sl], deg_hbm.at[sl])
                plsc.subcore_barrier()

        @pl.when(cid == 0)
        def _():
            run_set(e1_hbm, agg1_hbm, deg1_hbm)

        @pl.when(cid == 1)
        def _():
            run_set(e2_hbm, agg2_hbm, deg2_hbm)

    return k


def _sc_lookup(enc1, enc2, idx2d):
    NR, W = idx2d.shape
    V, H = enc1.shape
    rows_per_w = NR // 32
    mesh = plsc.VectorSubcoreMesh(core_axis_name="c", subcore_axis_name="s")

    @functools.partial(
        pl.kernel, mesh=mesh,
        out_type=jax.ShapeDtypeStruct((2 * NR * W, H), jnp.float32),
        scratch_types=[pltpu.VMEM((W,), jnp.int32),
                       pltpu.VMEM((W, H), jnp.float32),
                       pltpu.VMEM((W, H), jnp.float32),
                       pltpu.SemaphoreType.DMA,
                       pltpu.SemaphoreType.DMA],
    )
    def k(enc1_hbm, enc2_hbm, idx_hbm, out_hbm, idx_v, r1_v, r2_v, sem1, sem2):
        wid = lax.axis_index("s") * 2 + lax.axis_index("c")

        def body(r, _):
            row = wid * rows_per_w + r
            pltpu.sync_copy(idx_hbm.at[row], idx_v)
            cp1 = pltpu.async_copy(enc1_hbm.at[idx_v], r1_v, sem1)
            cp2 = pltpu.async_copy(enc2_hbm.at[idx_v], r2_v, sem2)
            cp1.wait()
            pltpu.sync_copy(r1_v, out_hbm.at[pl.ds(row * W, W)])
            cp2.wait()
            pltpu.sync_copy(r2_v, out_hbm.at[pl.ds((NR + row) * W, W)])
            return ()
        lax.fori_loop(0, rows_per_w, body, ())

    return k(enc1, enc2, idx2d)


def _gcn_finish_q_body(agg1, deg1, agg2, deg2, wg1, bg1, wg2, bg2, out1, out2):
    d1 = jnp.maximum(deg1[...], 1.0)
    d2 = jnp.maximum(deg2[...], 1.0)
    nq = agg1.shape[0]
    qw = agg1.shape[2]
    acc1 = bg1[...] * 1.0
    acc2 = bg2[...] * 1.0
    for q in range(nq):
        acc1 = acc1 + jnp.dot(agg1[q] / d1, wg1[pl.ds(q * qw, qw), :],
                              preferred_element_type=jnp.float32)
        acc2 = acc2 + jnp.dot(agg2[q] / d2, wg2[pl.ds(q * qw, qw), :],
                              preferred_element_type=jnp.float32)
    out1[...] = jnp.maximum(acc1, 0.0)
    out2[...] = jnp.maximum(acc2, 0.0)


def _gcn_finish_q(agg1q, deg1, agg2q, deg2, Wg1, bg1, Wg2, bg2, V, vb):
    nq, VP, qw = agg1q.shape
    D, H = Wg1.shape
    grid = (V // vb,)
    bs_a = pl.BlockSpec((nq, vb, qw), lambda i: (0, i, 0))
    bs_d = pl.BlockSpec((vb, 1), lambda i: (i, 0))
    bs_w = pl.BlockSpec((D, H), lambda i: (0, 0))
    bs_b = pl.BlockSpec((1, H), lambda i: (0, 0))
    return pl.pallas_call(
        _gcn_finish_q_body,
        grid=grid,
        in_specs=[bs_a, bs_d, bs_a, bs_d, bs_w, bs_b, bs_w, bs_b],
        out_specs=[pl.BlockSpec((vb, H), lambda i: (i, 0))] * 2,
        out_shape=[jax.ShapeDtypeStruct((V, H), jnp.float32)] * 2,
    )(agg1q, deg1[:, None], agg2q, deg2[:, None], Wg1, bg1[None], Wg2, bg2[None])


def _encoder_body(x_ref, padf_ref, wq, bq, wk, bk, wv, bv, wo, bo,
                  ln1g, ln1b, ln2g, ln2b, w1, b1, w2, b2, pooled_ref, *, bb, dh):
    sp = x_ref.shape[1]
    h = x_ref.shape[2]
    x = x_ref[...]
    x2 = x.reshape(bb * sp, h)
    padf = padf_ref[...]
    neg = padf * -1e9

    o_acc = jnp.zeros((bb * sp, h), jnp.float32)
    scale = 1.0 / np.sqrt(dh)
    nh = h // dh
    for hd in range(nh):
        wq_h = wq[:, hd * dh:(hd + 1) * dh]
        wk_h = wk[:, hd * dh:(hd + 1) * dh]
        wv_h = wv[:, hd * dh:(hd + 1) * dh]
        bq_h = bq[:, hd * dh:(hd + 1) * dh]
        bk_h = bk[:, hd * dh:(hd + 1) * dh]
        bv_h = bv[:, hd * dh:(hd + 1) * dh]
        q_h = (jnp.dot(x2, wq_h, preferred_element_type=jnp.float32) + bq_h
               ).reshape(bb, sp, dh)
        k_h = (jnp.dot(x2, wk_h, preferred_element_type=jnp.float32) + bk_h
               ).reshape(bb, sp, dh)
        v_h = (jnp.dot(x2, wv_h, preferred_element_type=jnp.float32) + bv_h
               ).reshape(bb, sp, dh)
        scores = jax.lax.dot_general(
            q_h, k_h, (((2,), (2,)), ((0,), (0,))),
            preferred_element_type=jnp.float32) * scale
        scores = scores + neg[:, None, :]
        scores = scores - jnp.max(scores, axis=-1, keepdims=True)
        e = jnp.exp(scores)
        attn = e / jnp.sum(e, axis=-1, keepdims=True)
        o_h = jax.lax.dot_general(
            attn, v_h, (((2,), (1,)), ((0,), (0,))),
            preferred_element_type=jnp.float32)
        wo_h = wo[hd * dh:(hd + 1) * dh, :]
        o_acc = o_acc + jnp.dot(o_h.reshape(bb * sp, dh), wo_h,
                                preferred_element_type=jnp.float32)
    o_acc = o_acc + bo[...]

    def ln(t, g, b):
        mu = jnp.mean(t, axis=-1, keepdims=True)
        var = jnp.mean((t - mu) ** 2, axis=-1, keepdims=True)
        return (t - mu) / jnp.sqrt(var + 1e-5) * g[...] + b[...]

    x2 = ln(x2 + o_acc, ln1g, ln1b)
    f = jnp.maximum(jnp.dot(x2, w1[...], preferred_element_type=jnp.float32)
                    + b1[...], 0.0)
    f = jnp.dot(f, w2[...], preferred_element_type=jnp.float32) + b2[...]
    x2 = ln(x2 + f, ln2g, ln2b)

    enc = x2.reshape(bb, sp, h)
    keep = (1.0 - padf)
    summed = jnp.sum(enc * keep[:, :, None], axis=1)
    cnt = jnp.sum(keep, axis=1, keepdims=True)
    pooled_ref[...] = summed / cnt


def _encoder_pool(emb_all, padf, p, bb):
    twob, sp, h = emb_all.shape
    b = twob // 2
    dh = h // NH
    nblk = twob // bb
    bpb = b // bb
    bs_x = pl.BlockSpec((bb, sp, h), lambda i: (i, 0, 0))
    bs_m = pl.BlockSpec((bb, sp), lambda i: (i % bpb, 0))
    full = lambda *shape: pl.BlockSpec(shape, lambda i: (0,) * len(shape))
    w = lambda a: full(*a.shape)
    body = functools.partial(_encoder_body, bb=bb, dh=dh)
    return pl.pallas_call(
        body,
        grid=(nblk,),
        in_specs=[bs_x, bs_m,
                  w(p['Wq']), full(1, h), w(p['Wk']), full(1, h),
                  w(p['Wv']), full(1, h), w(p['Wo']), full(1, h),
                  full(1, h), full(1, h), full(1, h), full(1, h),
                  w(p['W1']), full(1, p['W1'].shape[1]),
                  w(p['W2']), full(1, h)],
        out_specs=pl.BlockSpec((bb, h), lambda i: (i, 0)),
        out_shape=jax.ShapeDtypeStruct((twob, h), jnp.float32),
    )(emb_all, padf,
      p['Wq'], p['bq'][None], p['Wk'], p['bk'][None],
      p['Wv'], p['bv'][None], p['Wo'], p['bo'][None],
      p['ln1_g'][None], p['ln1_b'][None], p['ln2_g'][None], p['ln2_b'][None],
      p['W1'], p['b1'][None], p['W2'], p['b2'][None])


def kernel(sequences, edge_index1, edge_index2, node_table, Wg1, bg1, Wg2, bg2,
           Wq, bq, Wk, bk, Wv, bv, Wo, bo, ln1_g, ln1_b, ln2_g, ln2_b,
           W1, b1, W2, b2):
    V, D = node_table.shape
    B, S = sequences.shape
    H = Wg1.shape[1]
    E = edge_index1.shape[1]

    grp = 16 * CH * 4
    E_pad = ((E + grp - 1) // grp) * grp
    VP = ((V + 255) // 256) * 256
    npad = E_pad - E
    pad_src = (jnp.arange(npad, dtype=jnp.int32) % V)
    pad_dst = V + (jnp.arange(npad, dtype=jnp.int32) % (VP - V))
    pad_e = jnp.stack([pad_src, pad_dst])
    e1 = jnp.concatenate([edge_index1, pad_e], axis=1) \
        .reshape(2, E_pad // CH, CH).transpose(1, 0, 2)
    e2 = jnp.concatenate([edge_index2, pad_e], axis=1) \
        .reshape(2, E_pad // CH, CH).transpose(1, 0, 2)
    table_q = node_table.reshape(V, NQ, QW).transpose(1, 0, 2)

    seg_k = _sc_segsum(table_q, e1, e2, VP)
    agg1q, agg2q, deg1, deg2 = seg_k(table_q, e1, e2)

    vb = 2000 if V % 2000 == 0 else V
    node_enc1, node_enc2 = _gcn_finish_q(agg1q, deg1[:V], agg2q, deg2[:V],
                                         Wg1, bg1, Wg2, bg2, V, vb)

    sp = S_PAD if S <= S_PAD else S
    seq_pad = jnp.full((B, sp), V, jnp.int32).at[:, :S].set(sequences)
    padf = (seq_pad == V).astype(jnp.float32)
    flat = seq_pad.reshape(-1)
    fill = jnp.arange(flat.shape[0], dtype=jnp.int32) % V
    idx_eff = jnp.where(flat == V, fill, flat)
    idx2d = idx_eff.reshape(-1, 128)
    emb_flat = _sc_lookup(node_enc1, node_enc2, idx2d)
    emb_all = emb_flat.reshape(2 * B, sp, H)

    p = dict(Wq=Wq, bq=bq, Wk=Wk, bk=bk, Wv=Wv, bv=bv, Wo=Wo, bo=bo,
             ln1_g=ln1_g, ln1_b=ln1_b, ln2_g=ln2_g, ln2_b=ln2_b,
             W1=W1, b1=b1, W2=W2, b2=b2)
    bb = 64 if B % 64 == 0 else B
    pooled_all = _encoder_pool(emb_all, padf, p, bb)
    pooled1, pooled2 = pooled_all[:B], pooled_all[B:]
    return (node_enc1, node_enc2, pooled1, pooled2)


# speedup vs baseline: 5.6598x; 1.1848x over previous
import functools
from typing import Any

import jax
import jax.numpy as jnp
import numpy as np
from jax import lax
from jax.experimental import pallas as pl
from jax.experimental.pallas import tpu as pltpu
from jax.experimental.pallas import tpu_sc as plsc

NH = 4
S_PAD = 64
NQ = 4
QW = 32
CH = 128


def _sc_segsum(table_q, e1, e2, VP):
    NC = e1.shape[0]
    n_chunks = NC // 16
    per_tile = n_chunks
    stripe = VP // 16
    ZR = 224
    assert stripe % ZR == 0 and ZR % 16 == 0 and n_chunks % 4 == 0
    mesh = plsc.VectorSubcoreMesh(core_axis_name="c", subcore_axis_name="s")

    @functools.partial(
        pl.kernel, mesh=mesh,
        compiler_params=pltpu.CompilerParams(use_tc_tiling_on_sc=False),
        out_type=[jax.ShapeDtypeStruct((NQ, VP, QW), jnp.float32),
                  jax.ShapeDtypeStruct((NQ, VP, QW), jnp.float32),
                  jax.ShapeDtypeStruct((VP,), jnp.float32),
                  jax.ShapeDtypeStruct((VP,), jnp.float32)],
        scratch_types=[[pltpu.VMEM((2, CH), jnp.int32) for _ in range(4)],
                       [pltpu.VMEM((CH, QW), jnp.float32) for _ in range(2)],
                       pltpu.VMEM((CH,), jnp.float32),
                       pltpu.VMEM((ZR, QW), jnp.float32),
                       pltpu.VMEM((ZR,), jnp.float32),
                       [pltpu.SemaphoreType.DMA for _ in range(4)],
                       [pltpu.SemaphoreType.DMA for _ in range(2)],
                       [pltpu.SemaphoreType.DMA for _ in range(2)],
                       pltpu.SemaphoreType.DMA,
                       pltpu.VMEM_SHARED((VP, QW), jnp.float32),
                       pltpu.VMEM_SHARED((VP,), jnp.float32)],
    )
    def k(tq_hbm, e1_hbm, e2_hbm, agg1_hbm, agg2_hbm, deg1_hbm, deg2_hbm,
          ed_v, rows_v, ones_v, zrow_v, zdeg_v,
          sem_e, sem_g, sem_s, sem_deg, sc_shared, deg_shared):
        cid = lax.axis_index("c")
        tid = lax.axis_index("s")

        def fill2d(ref, n, val):
            def b(i, _):
                ref[i, pl.ds(0, 16)] = jnp.full((16,), val, jnp.float32)
                ref[i, pl.ds(16, 16)] = jnp.full((16,), val, jnp.float32)
                return ()
            lax.fori_loop(0, n, b, ())

        def fill1d(ref, n, val):
            def b(i, _):
                ref[pl.ds(i * 16, 16)] = jnp.full((16,), val, jnp.float32)
                return ()
            lax.fori_loop(0, n // 16, b, ())

        fill2d(zrow_v, ZR, 0.0)
        fill1d(zdeg_v, ZR, 0.0)
        fill1d(ones_v, CH, 1.0)

        def run_set(e_hbm, agg_hbm, deg_hbm):
            base = tid * per_tile
            n = n_chunks

            def wait_g(b):
                pltpu.make_async_copy(
                    tq_hbm.at[0].at[pl.ds(0, CH)], rows_v[b], sem_g[b]).wait()

            def wait_s(b):
                pltpu.make_async_copy(
                    rows_v[b], sc_shared.at[pl.ds(0, CH)], sem_s[b]).wait()

            for q in range(NQ):
                tq = tq_hbm.at[q]

                def zchunk(j, _):
                    pltpu.sync_copy(
                        zrow_v, sc_shared.at[pl.ds(tid * stripe + j * ZR, ZR)])
                    if q == 0:
                        pltpu.sync_copy(
                            zdeg_v,
                            deg_shared.at[pl.ds(tid * stripe + j * ZR, ZR)])
                    return ()
                lax.fori_loop(0, stripe // ZR, zchunk, ())
                plsc.subcore_barrier()

                pltpu.async_copy(e_hbm.at[base], ed_v[0], sem_e[0])
                pltpu.async_copy(e_hbm.at[base + 1], ed_v[1], sem_e[1])
                pltpu.make_async_copy(e_hbm.at[base], ed_v[0], sem_e[0]).wait()
                pltpu.async_copy(tq.at[ed_v[0].at[0]], rows_v[0], sem_g[0])

                def group(g4, _):
                    for j in range(4):
                        g = g4 * 4 + j
                        br, be = j % 2, j
                        @pl.when(g >= 1)
                        def _():
                            wait_s((j + 1) % 2)
                        @pl.when(g + 2 < n)
                        def _():
                            pltpu.async_copy(e_hbm.at[base + g + 2],
                                             ed_v[(j + 2) % 4], sem_e[(j + 2) % 4])
                        @pl.when(g + 1 < n)
                        def _():
                            pltpu.make_async_copy(
                                e_hbm.at[base], ed_v[(j + 1) % 4],
                                sem_e[(j + 1) % 4]).wait()
                            pltpu.async_copy(tq.at[ed_v[(j + 1) % 4].at[0]],
                                             rows_v[(j + 1) % 2],
                                             sem_g[(j + 1) % 2])
                        wait_g(br)
                        pltpu.async_copy(rows_v[br],
                                         sc_shared.at[ed_v[be].at[1]],
                                         sem_s[br], add=True)
                        if q == 0:
                            pltpu.async_copy(ones_v,
                                             deg_shared.at[ed_v[be].at[1]],
                                             sem_deg, add=True)
                    return ()
                lax.fori_loop(0, n // 4, group, ())

                wait_s((n - 1) % 2)
                if q == 0:
                    def dr(i, _):
                        pltpu.make_async_copy(
                            ones_v, deg_shared.at[pl.ds(0, CH)], sem_deg).wait()
                        return ()
                    lax.fori_loop(0, n, dr, ())
                plsc.subcore_barrier()

                sl = pl.ds(tid * stripe, stripe)
                pltpu.sync_copy(sc_shared.at[sl], agg_hbm.at[q].at[sl])
                if q == 0:
                    pltpu.sync_copy(deg_shared.at[sl], deg_hbm.at[sl])
                plsc.subcore_barrier()

        @pl.when(cid == 0)
        def _():
            run_set(e1_hbm, agg1_hbm, deg1_hbm)

        @pl.when(cid == 1)
        def _():
            run_set(e2_hbm, agg2_hbm, deg2_hbm)

    return k


def _sc_lookup(enc1, enc2, idx2d):
    NR, W = idx2d.shape
    V, H = enc1.shape
    rows_per_w = NR // 32
    mesh = plsc.VectorSubcoreMesh(core_axis_name="c", subcore_axis_name="s")

    @functools.partial(
        pl.kernel, mesh=mesh,
        out_type=jax.ShapeDtypeStruct((2 * NR * W, H), jnp.float32),
        scratch_types=[pltpu.VMEM((W,), jnp.int32),
                       pltpu.VMEM((W, H), jnp.float32),
                       pltpu.VMEM((W, H), jnp.float32),
                       pltpu.SemaphoreType.DMA,
                       pltpu.SemaphoreType.DMA],
    )
    def k(enc1_hbm, enc2_hbm, idx_hbm, out_hbm, idx_v, r1_v, r2_v, sem1, sem2):
        wid = lax.axis_index("s") * 2 + lax.axis_index("c")

        def body(r, _):
            row = wid * rows_per_w + r
            pltpu.sync_copy(idx_hbm.at[row], idx_v)
            cp1 = pltpu.async_copy(enc1_hbm.at[idx_v], r1_v, sem1)
            cp2 = pltpu.async_copy(enc2_hbm.at[idx_v], r2_v, sem2)
            cp1.wait()
            pltpu.sync_copy(r1_v, out_hbm.at[pl.ds(row * W, W)])
            cp2.wait()
            pltpu.sync_copy(r2_v, out_hbm.at[pl.ds((NR + row) * W, W)])
            return ()
        lax.fori_loop(0, rows_per_w, body, ())

    return k(enc1, enc2, idx2d)


def _gcn_finish_q_body(agg1, deg1, agg2, deg2, wg1, bg1, wg2, bg2, out1, out2):
    d1 = jnp.maximum(deg1[...], 1.0)
    d2 = jnp.maximum(deg2[...], 1.0)
    nq = agg1.shape[0]
    qw = agg1.shape[2]
    acc1 = bg1[...] * 1.0
    acc2 = bg2[...] * 1.0
    for q in range(nq):
        acc1 = acc1 + jnp.dot(agg1[q] / d1, wg1[pl.ds(q * qw, qw), :],
                              preferred_element_type=jnp.float32)
        acc2 = acc2 + jnp.dot(agg2[q] / d2, wg2[pl.ds(q * qw, qw), :],
                              preferred_element_type=jnp.float32)
    out1[...] = jnp.maximum(acc1, 0.0)
    out2[...] = jnp.maximum(acc2, 0.0)


def _gcn_finish_q(agg1q, deg1, agg2q, deg2, Wg1, bg1, Wg2, bg2, V, vb):
    nq, VP, qw = agg1q.shape
    D, H = Wg1.shape
    grid = (V // vb,)
    bs_a = pl.BlockSpec((nq, vb, qw), lambda i: (0, i, 0))
    bs_d = pl.BlockSpec((vb, 1), lambda i: (i, 0))
    bs_w = pl.BlockSpec((D, H), lambda i: (0, 0))
    bs_b = pl.BlockSpec((1, H), lambda i: (0, 0))
    return pl.pallas_call(
        _gcn_finish_q_body,
        grid=grid,
        in_specs=[bs_a, bs_d, bs_a, bs_d, bs_w, bs_b, bs_w, bs_b],
        out_specs=[pl.BlockSpec((vb, H), lambda i: (i, 0))] * 2,
        out_shape=[jax.ShapeDtypeStruct((V, H), jnp.float32)] * 2,
    )(agg1q, deg1[:, None], agg2q, deg2[:, None], Wg1, bg1[None], Wg2, bg2[None])


def _encoder_body(x_ref, padf_ref, wq, bq, wk, bk, wv, bv, wo, bo,
                  ln1g, ln1b, ln2g, ln2b, w1, b1, w2, b2, pooled_ref, *, bb, dh):
    sp = x_ref.shape[1]
    h = x_ref.shape[2]
    x = x_ref[...]
    x2 = x.reshape(bb * sp, h)
    padf = padf_ref[...]
    neg = padf * -1e9

    o_acc = jnp.zeros((bb * sp, h), jnp.float32)
    scale = 1.0 / np.sqrt(dh)
    nh = h // dh
    for hd in range(nh):
        wq_h = wq[:, hd * dh:(hd + 1) * dh]
        wk_h = wk[:, hd * dh:(hd + 1) * dh]
        wv_h = wv[:, hd * dh:(hd + 1) * dh]
        bq_h = bq[:, hd * dh:(hd + 1) * dh]
        bk_h = bk[:, hd * dh:(hd + 1) * dh]
        bv_h = bv[:, hd * dh:(hd + 1) * dh]
        q_h = (jnp.dot(x2, wq_h, preferred_element_type=jnp.float32) + bq_h
               ).reshape(bb, sp, dh)
        k_h = (jnp.dot(x2, wk_h, preferred_element_type=jnp.float32) + bk_h
               ).reshape(bb, sp, dh)
        v_h = (jnp.dot(x2, wv_h, preferred_element_type=jnp.float32) + bv_h
               ).reshape(bb, sp, dh)
        scores = jax.lax.dot_general(
            q_h, k_h, (((2,), (2,)), ((0,), (0,))),
            preferred_element_type=jnp.float32) * scale
        scores = scores + neg[:, None, :]
        scores = scores - jnp.max(scores, axis=-1, keepdims=True)
        e = jnp.exp(scores)
        attn = e / jnp.sum(e, axis=-1, keepdims=True)
        o_h = jax.lax.dot_general(
            attn, v_h, (((2,), (1,)), ((0,), (0,))),
            preferred_element_type=jnp.float32)
        wo_h = wo[hd * dh:(hd + 1) * dh, :]
        o_acc = o_acc + jnp.dot(o_h.reshape(bb * sp, dh), wo_h,
                                preferred_element_type=jnp.float32)
    o_acc = o_acc + bo[...]

    def ln(t, g, b):
        mu = jnp.mean(t, axis=-1, keepdims=True)
        var = jnp.mean((t - mu) ** 2, axis=-1, keepdims=True)
        return (t - mu) / jnp.sqrt(var + 1e-5) * g[...] + b[...]

    x2 = ln(x2 + o_acc, ln1g, ln1b)
    f = jnp.maximum(jnp.dot(x2, w1[...], preferred_element_type=jnp.float32)
                    + b1[...], 0.0)
    f = jnp.dot(f, w2[...], preferred_element_type=jnp.float32) + b2[...]
    x2 = ln(x2 + f, ln2g, ln2b)

    enc = x2.reshape(bb, sp, h)
    keep = (1.0 - padf)
    summed = jnp.sum(enc * keep[:, :, None], axis=1)
    cnt = jnp.sum(keep, axis=1, keepdims=True)
    pooled_ref[...] = summed / cnt


def _encoder_pool(emb_all, padf, p, bb):
    twob, sp, h = emb_all.shape
    b = twob // 2
    dh = h // NH
    nblk = twob // bb
    bpb = b // bb
    bs_x = pl.BlockSpec((bb, sp, h), lambda i: (i, 0, 0))
    bs_m = pl.BlockSpec((bb, sp), lambda i: (i % bpb, 0))
    full = lambda *shape: pl.BlockSpec(shape, lambda i: (0,) * len(shape))
    w = lambda a: full(*a.shape)
    body = functools.partial(_encoder_body, bb=bb, dh=dh)
    return pl.pallas_call(
        body,
        grid=(nblk,),
        in_specs=[bs_x, bs_m,
                  w(p['Wq']), full(1, h), w(p['Wk']), full(1, h),
                  w(p['Wv']), full(1, h), w(p['Wo']), full(1, h),
                  full(1, h), full(1, h), full(1, h), full(1, h),
                  w(p['W1']), full(1, p['W1'].shape[1]),
                  w(p['W2']), full(1, h)],
        out_specs=pl.BlockSpec((bb, h), lambda i: (i, 0)),
        out_shape=jax.ShapeDtypeStruct((twob, h), jnp.float32),
    )(emb_all, padf,
      p['Wq'], p['bq'][None], p['Wk'], p['bk'][None],
      p['Wv'], p['bv'][None], p['Wo'], p['bo'][None],
      p['ln1_g'][None], p['ln1_b'][None], p['ln2_g'][None], p['ln2_b'][None],
      p['W1'], p['b1'][None], p['W2'], p['b2'][None])


def kernel(sequences, edge_index1, edge_index2, node_table, Wg1, bg1, Wg2, bg2,
           Wq, bq, Wk, bk, Wv, bv, Wo, bo, ln1_g, ln1_b, ln2_g, ln2_b,
           W1, b1, W2, b2):
    V, D = node_table.shape
    B, S = sequences.shape
    H = Wg1.shape[1]
    E = edge_index1.shape[1]

    grp = 16 * CH * 4
    E_pad = ((E + grp - 1) // grp) * grp
    VP = ((V + 255) // 256) * 256
    npad = E_pad - E
    pad_src = (jnp.arange(npad, dtype=jnp.int32) % V)
    pad_dst = V + (jnp.arange(npad, dtype=jnp.int32) % (VP - V))
    pad_e = jnp.stack([pad_src, pad_dst])
    e1 = jnp.concatenate([edge_index1, pad_e], axis=1) \
        .reshape(2, E_pad // CH, CH).transpose(1, 0, 2)
    e2 = jnp.concatenate([edge_index2, pad_e], axis=1) \
        .reshape(2, E_pad // CH, CH).transpose(1, 0, 2)
    table_q = node_table.reshape(V, NQ, QW).transpose(1, 0, 2)

    seg_k = _sc_segsum(table_q, e1, e2, VP)
    agg1q, agg2q, deg1, deg2 = seg_k(table_q, e1, e2)

    vb = 2000 if V % 2000 == 0 else V
    node_enc1, node_enc2 = _gcn_finish_q(agg1q, deg1[:V], agg2q, deg2[:V],
                                         Wg1, bg1, Wg2, bg2, V, vb)

    sp = S_PAD if S <= S_PAD else S
    seq_pad = jnp.full((B, sp), V, jnp.int32).at[:, :S].set(sequences)
    padf = (seq_pad == V).astype(jnp.float32)
    flat = seq_pad.reshape(-1)
    fill = jnp.arange(flat.shape[0], dtype=jnp.int32) % V
    idx_eff = jnp.where(flat == V, fill, flat)
    idx2d = idx_eff.reshape(-1, 128)
    emb_flat = _sc_lookup(node_enc1, node_enc2, idx2d)
    emb_all = emb_flat.reshape(2 * B, sp, H)

    p = dict(Wq=Wq, bq=bq, Wk=Wk, bk=bk, Wv=Wv, bv=bv, Wo=Wo, bo=bo,
             ln1_g=ln1_g, ln1_b=ln1_b, ln2_g=ln2_g, ln2_b=ln2_b,
             W1=W1, b1=b1, W2=W2, b2=b2)
    bb = 64 if B % 64 == 0 else B
    pooled_all = _encoder_pool(emb_all, padf, p, bb)
    pooled1, pooled2 = pooled_all[:B], pooled_all[B:]
    return (node_enc1, node_enc2, pooled1, pooled2)


# speedup vs baseline: 5.8445x; 1.0326x over previous
import functools
from typing import Any

import jax
import jax.numpy as jnp
import numpy as np
from jax import lax
from jax.experimental import pallas as pl
from jax.experimental.pallas import tpu as pltpu
from jax.experimental.pallas import tpu_sc as plsc

NH = 4
S_PAD = 64
NQ = 4
QW = 32
CH = 128


def _sc_segsum(table_q, e1, e2, VP):
    NC = e1.shape[0]
    n_chunks = NC // 16
    per_tile = n_chunks
    stripe = VP // 16
    ZR = 224
    assert stripe % ZR == 0 and ZR % 16 == 0 and n_chunks % 4 == 0
    mesh = plsc.VectorSubcoreMesh(core_axis_name="c", subcore_axis_name="s")

    @functools.partial(
        pl.kernel, mesh=mesh,
        compiler_params=pltpu.CompilerParams(use_tc_tiling_on_sc=False),
        out_type=[jax.ShapeDtypeStruct((NQ, VP, QW), jnp.float32),
                  jax.ShapeDtypeStruct((NQ, VP, QW), jnp.float32),
                  jax.ShapeDtypeStruct((VP,), jnp.float32),
                  jax.ShapeDtypeStruct((VP,), jnp.float32)],
        scratch_types=[[pltpu.VMEM((2, CH), jnp.int32) for _ in range(4)],
                       [pltpu.VMEM((CH, QW), jnp.float32) for _ in range(2)],
                       pltpu.VMEM((CH,), jnp.float32),
                       pltpu.VMEM((ZR, QW), jnp.float32),
                       pltpu.VMEM((ZR,), jnp.float32),
                       [pltpu.SemaphoreType.DMA for _ in range(4)],
                       [pltpu.SemaphoreType.DMA for _ in range(2)],
                       [pltpu.SemaphoreType.DMA for _ in range(2)],
                       pltpu.SemaphoreType.DMA,
                       pltpu.VMEM_SHARED((VP, QW), jnp.float32),
                       pltpu.VMEM_SHARED((VP,), jnp.float32)],
    )
    def k(tq_hbm, e1_hbm, e2_hbm, agg1_hbm, agg2_hbm, deg1_hbm, deg2_hbm,
          ed_v, rows_v, ones_v, zrow_v, zdeg_v,
          sem_e, sem_g, sem_s, sem_deg, sc_shared, deg_shared):
        cid = lax.axis_index("c")
        tid = lax.axis_index("s")

        def fill2d(ref, n, val):
            def b(i, _):
                ref[i, pl.ds(0, 16)] = jnp.full((16,), val, jnp.float32)
                ref[i, pl.ds(16, 16)] = jnp.full((16,), val, jnp.float32)
                return ()
            lax.fori_loop(0, n, b, ())

        def fill1d(ref, n, val):
            def b(i, _):
                ref[pl.ds(i * 16, 16)] = jnp.full((16,), val, jnp.float32)
                return ()
            lax.fori_loop(0, n // 16, b, ())

        fill2d(zrow_v, ZR, 0.0)
        fill1d(zdeg_v, ZR, 0.0)
        fill1d(ones_v, CH, 1.0)

        def run_set(e_hbm, agg_hbm, deg_hbm):
            base = tid * per_tile
            n = n_chunks

            def wait_g(b):
                pltpu.make_async_copy(
                    tq_hbm.at[0].at[pl.ds(0, CH)], rows_v[b], sem_g[b]).wait()

            def wait_s(b):
                pltpu.make_async_copy(
                    rows_v[b], sc_shared.at[pl.ds(0, CH)], sem_s[b]).wait()

            for q in range(NQ):
                tq = tq_hbm.at[q]

                def zchunk(j, _):
                    pltpu.sync_copy(
                        zrow_v, sc_shared.at[pl.ds(tid * stripe + j * ZR, ZR)])
                    if q == 0:
                        pltpu.sync_copy(
                            zdeg_v,
                            deg_shared.at[pl.ds(tid * stripe + j * ZR, ZR)])
                    return ()
                lax.fori_loop(0, stripe // ZR, zchunk, ())
                plsc.subcore_barrier()

                pltpu.async_copy(e_hbm.at[base], ed_v[0], sem_e[0])
                pltpu.async_copy(e_hbm.at[base + 1], ed_v[1], sem_e[1])
                pltpu.make_async_copy(e_hbm.at[base], ed_v[0], sem_e[0]).wait()
                pltpu.async_copy(tq.at[ed_v[0].at[0]], rows_v[0], sem_g[0])

                def group(g4, _):
                    for j in range(4):
                        g = g4 * 4 + j
                        br, be = j % 2, j
                        @pl.when(g >= 1)
                        def _():
                            wait_s((j + 1) % 2)
                        @pl.when(g + 2 < n)
                        def _():
                            pltpu.async_copy(e_hbm.at[base + g + 2],
                                             ed_v[(j + 2) % 4], sem_e[(j + 2) % 4])
                        @pl.when(g + 1 < n)
                        def _():
                            pltpu.make_async_copy(
                                e_hbm.at[base], ed_v[(j + 1) % 4],
                                sem_e[(j + 1) % 4]).wait()
                            pltpu.async_copy(tq.at[ed_v[(j + 1) % 4].at[0]],
                                             rows_v[(j + 1) % 2],
                                             sem_g[(j + 1) % 2])
                        wait_g(br)
                        pltpu.async_copy(rows_v[br],
                                         sc_shared.at[ed_v[be].at[1]],
                                         sem_s[br], add=True)
                        if q == 0:
                            pltpu.async_copy(ones_v,
                                             deg_shared.at[ed_v[be].at[1]],
                                             sem_deg, add=True)
                    return ()
                lax.fori_loop(0, n // 4, group, ())

                wait_s((n - 1) % 2)
                if q == 0:
                    def dr(i, _):
                        pltpu.make_async_copy(
                            ones_v, deg_shared.at[pl.ds(0, CH)], sem_deg).wait()
                        return ()
                    lax.fori_loop(0, n, dr, ())
                plsc.subcore_barrier()

                sl = pl.ds(tid * stripe, stripe)
                pltpu.sync_copy(sc_shared.at[sl], agg_hbm.at[q].at[sl])
                if q == 0:
                    pltpu.sync_copy(deg_shared.at[sl], deg_hbm.at[sl])
                plsc.subcore_barrier()

        @pl.when(cid == 0)
        def _():
            run_set(e1_hbm, agg1_hbm, deg1_hbm)

        @pl.when(cid == 1)
        def _():
            run_set(e2_hbm, agg2_hbm, deg2_hbm)

    return k


def _sc_lookup(enc1, enc2, idx2d):
    NR, W = idx2d.shape
    V, H = enc1.shape
    rows_per_w = NR // 32
    mesh = plsc.VectorSubcoreMesh(core_axis_name="c", subcore_axis_name="s")

    @functools.partial(
        pl.kernel, mesh=mesh,
        out_type=jax.ShapeDtypeStruct((2 * NR * W, H), jnp.float32),
        scratch_types=[pltpu.VMEM((W,), jnp.int32),
                       pltpu.VMEM((W, H), jnp.float32),
                       pltpu.VMEM((W, H), jnp.float32),
                       pltpu.SemaphoreType.DMA,
                       pltpu.SemaphoreType.DMA],
    )
    def k(enc1_hbm, enc2_hbm, idx_hbm, out_hbm, idx_v, r1_v, r2_v, sem1, sem2):
        wid = lax.axis_index("s") * 2 + lax.axis_index("c")

        def body(r, _):
            row = wid * rows_per_w + r
            pltpu.sync_copy(idx_hbm.at[row], idx_v)
            cp1 = pltpu.async_copy(enc1_hbm.at[idx_v], r1_v, sem1)
            cp2 = pltpu.async_copy(enc2_hbm.at[idx_v], r2_v, sem2)
            cp1.wait()
            pltpu.sync_copy(r1_v, out_hbm.at[pl.ds(row * W, W)])
            cp2.wait()
            pltpu.sync_copy(r2_v, out_hbm.at[pl.ds((NR + row) * W, W)])
            return ()
        lax.fori_loop(0, rows_per_w, body, ())

    return k(enc1, enc2, idx2d)


def _gcn_finish_q_body(agg1, deg1, agg2, deg2, wg1, bg1, wg2, bg2, out1, out2):
    d1 = jnp.maximum(deg1[...], 1.0)
    d2 = jnp.maximum(deg2[...], 1.0)
    nq = agg1.shape[0]
    qw = agg1.shape[2]
    acc1 = bg1[...] * 1.0
    acc2 = bg2[...] * 1.0
    for q in range(nq):
        acc1 = acc1 + jnp.dot(agg1[q] / d1, wg1[pl.ds(q * qw, qw), :],
                              preferred_element_type=jnp.float32)
        acc2 = acc2 + jnp.dot(agg2[q] / d2, wg2[pl.ds(q * qw, qw), :],
                              preferred_element_type=jnp.float32)
    out1[...] = jnp.maximum(acc1, 0.0)
    out2[...] = jnp.maximum(acc2, 0.0)


def _gcn_finish_q(agg1q, deg1, agg2q, deg2, Wg1, bg1, Wg2, bg2, V, vb):
    nq, VP, qw = agg1q.shape
    D, H = Wg1.shape
    grid = (V // vb,)
    bs_a = pl.BlockSpec((nq, vb, qw), lambda i: (0, i, 0))
    bs_d = pl.BlockSpec((vb, 1), lambda i: (i, 0))
    bs_w = pl.BlockSpec((D, H), lambda i: (0, 0))
    bs_b = pl.BlockSpec((1, H), lambda i: (0, 0))
    return pl.pallas_call(
        _gcn_finish_q_body,
        grid=grid,
        in_specs=[bs_a, bs_d, bs_a, bs_d, bs_w, bs_b, bs_w, bs_b],
        out_specs=[pl.BlockSpec((vb, H), lambda i: (i, 0))] * 2,
        out_shape=[jax.ShapeDtypeStruct((V, H), jnp.float32)] * 2,
    )(agg1q, deg1[:, None], agg2q, deg2[:, None], Wg1, bg1[None], Wg2, bg2[None])


def _encoder_body(x_ref, padf_ref, wq, bq, wk, bk, wv, bv, wo, bo,
                  ln1g, ln1b, ln2g, ln2b, w1, b1, w2, b2, pooled_ref, *, bb, dh):
    sp = x_ref.shape[1]
    h = x_ref.shape[2]
    bf = jnp.bfloat16
    x = x_ref[...]
    x2 = x.reshape(bb * sp, h)
    x2b = x2.astype(bf)
    padf = padf_ref[...]
    neg = padf * -1e9

    o_acc = jnp.zeros((bb * sp, h), jnp.float32)
    scale = 1.0 / np.sqrt(dh)
    nh = h // dh
    for hd in range(nh):
        wq_h = wq[:, hd * dh:(hd + 1) * dh].astype(bf)
        wk_h = wk[:, hd * dh:(hd + 1) * dh].astype(bf)
        wv_h = wv[:, hd * dh:(hd + 1) * dh].astype(bf)
        bq_h = bq[:, hd * dh:(hd + 1) * dh]
        bk_h = bk[:, hd * dh:(hd + 1) * dh]
        bv_h = bv[:, hd * dh:(hd + 1) * dh]
        q_h = (jnp.dot(x2b, wq_h, preferred_element_type=jnp.float32) + bq_h
               ).reshape(bb, sp, dh)
        k_h = (jnp.dot(x2b, wk_h, preferred_element_type=jnp.float32) + bk_h
               ).reshape(bb, sp, dh)
        v_h = (jnp.dot(x2b, wv_h, preferred_element_type=jnp.float32) + bv_h
               ).reshape(bb, sp, dh)
        scores = jax.lax.dot_general(
            q_h, k_h, (((2,), (2,)), ((0,), (0,))),
            preferred_element_type=jnp.float32) * scale
        e = jnp.exp(scores + neg[:, None, :])
        attn = (e / jnp.sum(e, axis=-1, keepdims=True)).astype(bf)
        o_h = jax.lax.dot_general(
            attn, v_h.astype(bf), (((2,), (1,)), ((0,), (0,))),
            preferred_element_type=jnp.float32)
        wo_h = wo[hd * dh:(hd + 1) * dh, :].astype(bf)
        o_acc = o_acc + jnp.dot(o_h.reshape(bb * sp, dh).astype(bf), wo_h,
                                preferred_element_type=jnp.float32)
    o_acc = o_acc + bo[...]

    def ln(t, g, b):
        mu = jnp.mean(t, axis=-1, keepdims=True)
        var = jnp.mean((t - mu) ** 2, axis=-1, keepdims=True)
        return (t - mu) / jnp.sqrt(var + 1e-5) * g[...] + b[...]

    x2 = ln(x2 + o_acc, ln1g, ln1b)
    f = jnp.maximum(jnp.dot(x2.astype(bf), w1[...].astype(bf),
                            preferred_element_type=jnp.float32)
                    + b1[...], 0.0)
    f = jnp.dot(f.astype(bf), w2[...].astype(bf),
                preferred_element_type=jnp.float32) + b2[...]
    x2 = ln(x2 + f, ln2g, ln2b)

    enc = x2.reshape(bb, sp, h)
    keep = (1.0 - padf)
    summed = jnp.sum(enc * keep[:, :, None], axis=1)
    cnt = jnp.sum(keep, axis=1, keepdims=True)
    pooled_ref[...] = summed / cnt


def _encoder_pool(emb_all, padf, p, bb):
    twob, sp, h = emb_all.shape
    b = twob // 2
    dh = h // NH
    nblk = twob // bb
    bpb = b // bb
    bs_x = pl.BlockSpec((bb, sp, h), lambda i: (i, 0, 0))
    bs_m = pl.BlockSpec((bb, sp), lambda i: (i % bpb, 0))
    full = lambda *shape: pl.BlockSpec(shape, lambda i: (0,) * len(shape))
    w = lambda a: full(*a.shape)
    body = functools.partial(_encoder_body, bb=bb, dh=dh)
    return pl.pallas_call(
        body,
        grid=(nblk,),
        in_specs=[bs_x, bs_m,
                  w(p['Wq']), full(1, h), w(p['Wk']), full(1, h),
                  w(p['Wv']), full(1, h), w(p['Wo']), full(1, h),
                  full(1, h), full(1, h), full(1, h), full(1, h),
                  w(p['W1']), full(1, p['W1'].shape[1]),
                  w(p['W2']), full(1, h)],
        out_specs=pl.BlockSpec((bb, h), lambda i: (i, 0)),
        out_shape=jax.ShapeDtypeStruct((twob, h), jnp.float32),
    )(emb_all, padf,
      p['Wq'], p['bq'][None], p['Wk'], p['bk'][None],
      p['Wv'], p['bv'][None], p['Wo'], p['bo'][None],
      p['ln1_g'][None], p['ln1_b'][None], p['ln2_g'][None], p['ln2_b'][None],
      p['W1'], p['b1'][None], p['W2'], p['b2'][None])


def kernel(sequences, edge_index1, edge_index2, node_table, Wg1, bg1, Wg2, bg2,
           Wq, bq, Wk, bk, Wv, bv, Wo, bo, ln1_g, ln1_b, ln2_g, ln2_b,
           W1, b1, W2, b2):
    V, D = node_table.shape
    B, S = sequences.shape
    H = Wg1.shape[1]
    E = edge_index1.shape[1]

    grp = 16 * CH * 4
    E_pad = ((E + grp - 1) // grp) * grp
    VP = ((V + 255) // 256) * 256
    npad = E_pad - E
    pad_src = (jnp.arange(npad, dtype=jnp.int32) % V)
    pad_dst = V + (jnp.arange(npad, dtype=jnp.int32) % (VP - V))
    pad_e = jnp.stack([pad_src, pad_dst])
    e1 = jnp.concatenate([edge_index1, pad_e], axis=1) \
        .reshape(2, E_pad // CH, CH).transpose(1, 0, 2)
    e2 = jnp.concatenate([edge_index2, pad_e], axis=1) \
        .reshape(2, E_pad // CH, CH).transpose(1, 0, 2)
    table_q = node_table.reshape(V, NQ, QW).transpose(1, 0, 2)

    seg_k = _sc_segsum(table_q, e1, e2, VP)
    agg1q, agg2q, deg1, deg2 = seg_k(table_q, e1, e2)

    vb = 2000 if V % 2000 == 0 else V
    node_enc1, node_enc2 = _gcn_finish_q(agg1q, deg1[:V], agg2q, deg2[:V],
                                         Wg1, bg1, Wg2, bg2, V, vb)

    sp = S_PAD if S <= S_PAD else S
    seq_pad = jnp.full((B, sp), V, jnp.int32).at[:, :S].set(sequences)
    padf = (seq_pad == V).astype(jnp.float32)
    flat = seq_pad.reshape(-1)
    fill = jnp.arange(flat.shape[0], dtype=jnp.int32) % V
    idx_eff = jnp.where(flat == V, fill, flat)
    idx2d = idx_eff.reshape(-1, 128)
    emb_flat = _sc_lookup(node_enc1, node_enc2, idx2d)
    emb_all = emb_flat.reshape(2 * B, sp, H)

    p = dict(Wq=Wq, bq=bq, Wk=Wk, bk=bk, Wv=Wv, bv=bv, Wo=Wo, bo=bo,
             ln1_g=ln1_g, ln1_b=ln1_b, ln2_g=ln2_g, ln2_b=ln2_b,
             W1=W1, b1=b1, W2=W2, b2=b2)
    bb = 64 if B % 64 == 0 else B
    pooled_all = _encoder_pool(emb_all, padf, p, bb)
    pooled1, pooled2 = pooled_all[:B], pooled_all[B:]
    return (node_enc1, node_enc2, pooled1, pooled2)


# speedup vs baseline: 6.9747x; 1.1934x over previous
import functools
from typing import Any

import jax
import jax.numpy as jnp
import numpy as np
from jax import lax
from jax.experimental import pallas as pl
from jax.experimental.pallas import tpu as pltpu
from jax.experimental.pallas import tpu_sc as plsc

NH = 4
S_PAD = 64
NQ = 4
QW = 32
CH = 128


def _sc_segsum(table_q, e1, e2, VP):
    NC = e1.shape[0]
    n_chunks = NC // 16
    per_tile = n_chunks
    nl = n_chunks // 2
    stripe = VP // 16
    ZR = 112
    assert stripe % ZR == 0 and ZR % 16 == 0 and nl % 4 == 0
    mesh = plsc.VectorSubcoreMesh(core_axis_name="c", subcore_axis_name="s")

    @functools.partial(
        pl.kernel, mesh=mesh,
        compiler_params=pltpu.CompilerParams(use_tc_tiling_on_sc=False),
        out_type=[jax.ShapeDtypeStruct((NQ, VP, QW), jnp.float32),
                  jax.ShapeDtypeStruct((NQ, VP, QW), jnp.float32),
                  jax.ShapeDtypeStruct((VP,), jnp.float32),
                  jax.ShapeDtypeStruct((VP,), jnp.float32)],
        scratch_types=[[[pltpu.VMEM((2, CH), jnp.int32) for _ in range(4)]
                        for _ in range(2)],
                       [[pltpu.VMEM((CH, QW), jnp.float32) for _ in range(2)]
                        for _ in range(2)],
                       pltpu.VMEM((CH,), jnp.float32),
                       pltpu.VMEM((ZR, QW), jnp.float32),
                       pltpu.VMEM((ZR,), jnp.float32),
                       [[pltpu.SemaphoreType.DMA for _ in range(4)]
                        for _ in range(2)],
                       [[pltpu.SemaphoreType.DMA for _ in range(2)]
                        for _ in range(2)],
                       [[pltpu.SemaphoreType.DMA for _ in range(2)]
                        for _ in range(2)],
                       pltpu.SemaphoreType.DMA,
                       pltpu.VMEM_SHARED((VP, QW), jnp.float32),
                       pltpu.VMEM_SHARED((VP,), jnp.float32)],
    )
    def k(tq_hbm, e1_hbm, e2_hbm, agg1_hbm, agg2_hbm, deg1_hbm, deg2_hbm,
          ed_v, rows_v, ones_v, zrow_v, zdeg_v,
          sem_e, sem_g, sem_s, sem_deg, sc_shared, deg_shared):
        cid = lax.axis_index("c")
        tid = lax.axis_index("s")

        def fill2d(ref, n, val):
            def b(i, _):
                ref[i, pl.ds(0, 16)] = jnp.full((16,), val, jnp.float32)
                ref[i, pl.ds(16, 16)] = jnp.full((16,), val, jnp.float32)
                return ()
            lax.fori_loop(0, n, b, ())

        def fill1d(ref, n, val):
            def b(i, _):
                ref[pl.ds(i * 16, 16)] = jnp.full((16,), val, jnp.float32)
                return ()
            lax.fori_loop(0, n // 16, b, ())

        fill2d(zrow_v, ZR, 0.0)
        fill1d(zdeg_v, ZR, 0.0)
        fill1d(ones_v, CH, 1.0)

        def run_set(e_hbm, agg_hbm, deg_hbm):
            base = tid * per_tile
            n = nl

            def wait_g(L, b):
                pltpu.make_async_copy(
                    tq_hbm.at[0].at[pl.ds(0, CH)], rows_v[L][b],
                    sem_g[L][b]).wait()

            def wait_s(L, b):
                pltpu.make_async_copy(
                    rows_v[L][b], sc_shared.at[pl.ds(0, CH)],
                    sem_s[L][b]).wait()

            for q in range(NQ):
                tq = tq_hbm.at[q]

                def zchunk(j, _):
                    pltpu.sync_copy(
                        zrow_v, sc_shared.at[pl.ds(tid * stripe + j * ZR, ZR)])
                    if q == 0:
                        pltpu.sync_copy(
                            zdeg_v,
                            deg_shared.at[pl.ds(tid * stripe + j * ZR, ZR)])
                    return ()
                lax.fori_loop(0, stripe // ZR, zchunk, ())
                plsc.subcore_barrier()

                def cidx(L, t):
                    return base + 2 * t + L

                for L in range(2):
                    pltpu.async_copy(e_hbm.at[cidx(L, 0)], ed_v[L][0],
                                     sem_e[L][0])
                    pltpu.async_copy(e_hbm.at[cidx(L, 1)], ed_v[L][1],
                                     sem_e[L][1])
                for L in range(2):
                    pltpu.make_async_copy(e_hbm.at[cidx(L, 0)], ed_v[L][0],
                                          sem_e[L][0]).wait()
                    pltpu.async_copy(tq.at[ed_v[L][0].at[0]], rows_v[L][0],
                                     sem_g[L][0])

                def step(L, g, j):
                    br, be = j % 2, j % 4
                    @pl.when(g >= 1)
                    def _():
                        wait_s(L, (j + 1) % 2)
                    @pl.when(g + 2 < n)
                    def _():
                        pltpu.async_copy(e_hbm.at[cidx(L, g + 2)],
                                         ed_v[L][(j + 2) % 4],
                                         sem_e[L][(j + 2) % 4])
                    @pl.when(g + 1 < n)
                    def _():
                        pltpu.make_async_copy(
                            e_hbm.at[cidx(L, 0)], ed_v[L][(j + 1) % 4],
                            sem_e[L][(j + 1) % 4]).wait()
                        pltpu.async_copy(tq.at[ed_v[L][(j + 1) % 4].at[0]],
                                         rows_v[L][(j + 1) % 2],
                                         sem_g[L][(j + 1) % 2])
                    wait_g(L, br)
                    pltpu.async_copy(rows_v[L][br],
                                     sc_shared.at[ed_v[L][be].at[1]],
                                     sem_s[L][br], add=True)
                    if q == 0:
                        pltpu.async_copy(ones_v,
                                         deg_shared.at[ed_v[L][be].at[1]],
                                         sem_deg, add=True)

                def group(g4, _):
                    for j in range(4):
                        g = g4 * 4 + j
                        step(0, g, j)
                        step(1, g, j)
                    return ()
                lax.fori_loop(0, n // 4, group, ())

                wait_s(0, (n - 1) % 2)
                wait_s(1, (n - 1) % 2)
                if q == 0:
                    def dr(i, _):
                        pltpu.make_async_copy(
                            ones_v, deg_shared.at[pl.ds(0, CH)], sem_deg).wait()
                        return ()
                    lax.fori_loop(0, 2 * n, dr, ())
                plsc.subcore_barrier()

                sl = pl.ds(tid * stripe, stripe)
                pltpu.sync_copy(sc_shared.at[sl], agg_hbm.at[q].at[sl])
                if q == 0:
                    pltpu.sync_copy(deg_shared.at[sl], deg_hbm.at[sl])
                plsc.subcore_barrier()

        @pl.when(cid == 0)
        def _():
            run_set(e1_hbm, agg1_hbm, deg1_hbm)

        @pl.when(cid == 1)
        def _():
            run_set(e2_hbm, agg2_hbm, deg2_hbm)

    return k


def _sc_lookup(enc1, enc2, idx2d):
    NR, W = idx2d.shape
    V, H = enc1.shape
    rows_per_w = NR // 32
    mesh = plsc.VectorSubcoreMesh(core_axis_name="c", subcore_axis_name="s")

    @functools.partial(
        pl.kernel, mesh=mesh,
        out_type=jax.ShapeDtypeStruct((2 * NR * W, H), jnp.float32),
        scratch_types=[pltpu.VMEM((W,), jnp.int32),
                       pltpu.VMEM((W, H), jnp.float32),
                       pltpu.VMEM((W, H), jnp.float32),
                       pltpu.SemaphoreType.DMA,
                       pltpu.SemaphoreType.DMA],
    )
    def k(enc1_hbm, enc2_hbm, idx_hbm, out_hbm, idx_v, r1_v, r2_v, sem1, sem2):
        wid = lax.axis_index("s") * 2 + lax.axis_index("c")

        def body(r, _):
            row = wid * rows_per_w + r
            pltpu.sync_copy(idx_hbm.at[row], idx_v)
            cp1 = pltpu.async_copy(enc1_hbm.at[idx_v], r1_v, sem1)
            cp2 = pltpu.async_copy(enc2_hbm.at[idx_v], r2_v, sem2)
            cp1.wait()
            pltpu.sync_copy(r1_v, out_hbm.at[pl.ds(row * W, W)])
            cp2.wait()
            pltpu.sync_copy(r2_v, out_hbm.at[pl.ds((NR + row) * W, W)])
            return ()
        lax.fori_loop(0, rows_per_w, body, ())

    return k(enc1, enc2, idx2d)


def _gcn_finish_q_body(agg1, deg1, agg2, deg2, wg1, bg1, wg2, bg2, out1, out2):
    d1 = jnp.maximum(deg1[...], 1.0)
    d2 = jnp.maximum(deg2[...], 1.0)
    nq = agg1.shape[0]
    qw = agg1.shape[2]
    acc1 = bg1[...] * 1.0
    acc2 = bg2[...] * 1.0
    for q in range(nq):
        acc1 = acc1 + jnp.dot(agg1[q] / d1, wg1[pl.ds(q * qw, qw), :],
                              preferred_element_type=jnp.float32)
        acc2 = acc2 + jnp.dot(agg2[q] / d2, wg2[pl.ds(q * qw, qw), :],
                              preferred_element_type=jnp.float32)
    out1[...] = jnp.maximum(acc1, 0.0)
    out2[...] = jnp.maximum(acc2, 0.0)


def _gcn_finish_q(agg1q, deg1, agg2q, deg2, Wg1, bg1, Wg2, bg2, V, vb):
    nq, VP, qw = agg1q.shape
    D, H = Wg1.shape
    grid = (V // vb,)
    bs_a = pl.BlockSpec((nq, vb, qw), lambda i: (0, i, 0))
    bs_d = pl.BlockSpec((vb, 1), lambda i: (i, 0))
    bs_w = pl.BlockSpec((D, H), lambda i: (0, 0))
    bs_b = pl.BlockSpec((1, H), lambda i: (0, 0))
    return pl.pallas_call(
        _gcn_finish_q_body,
        grid=grid,
        in_specs=[bs_a, bs_d, bs_a, bs_d, bs_w, bs_b, bs_w, bs_b],
        out_specs=[pl.BlockSpec((vb, H), lambda i: (i, 0))] * 2,
        out_shape=[jax.ShapeDtypeStruct((V, H), jnp.float32)] * 2,
    )(agg1q, deg1[:, None], agg2q, deg2[:, None], Wg1, bg1[None], Wg2, bg2[None])


def _encoder_body(x_ref, padf_ref, wq, bq, wk, bk, wv, bv, wo, bo,
                  ln1g, ln1b, ln2g, ln2b, w1, b1, w2, b2, pooled_ref, *, bb, dh):
    sp = x_ref.shape[1]
    h = x_ref.shape[2]
    bf = jnp.bfloat16
    x = x_ref[...]
    x2 = x.reshape(bb * sp, h)
    x2b = x2.astype(bf)
    padf = padf_ref[...]
    neg = padf * -1e9

    o_acc = jnp.zeros((bb * sp, h), jnp.float32)
    scale = 1.0 / np.sqrt(dh)
    nh = h // dh
    for hd in range(nh):
        wq_h = wq[:, hd * dh:(hd + 1) * dh].astype(bf)
        wk_h = wk[:, hd * dh:(hd + 1) * dh].astype(bf)
        wv_h = wv[:, hd * dh:(hd + 1) * dh].astype(bf)
        bq_h = bq[:, hd * dh:(hd + 1) * dh]
        bk_h = bk[:, hd * dh:(hd + 1) * dh]
        bv_h = bv[:, hd * dh:(hd + 1) * dh]
        q_h = (jnp.dot(x2b, wq_h, preferred_element_type=jnp.float32) + bq_h
               ).reshape(bb, sp, dh)
        k_h = (jnp.dot(x2b, wk_h, preferred_element_type=jnp.float32) + bk_h
               ).reshape(bb, sp, dh)
        v_h = (jnp.dot(x2b, wv_h, preferred_element_type=jnp.float32) + bv_h
               ).reshape(bb, sp, dh)
        scores = jax.lax.dot_general(
            q_h, k_h, (((2,), (2,)), ((0,), (0,))),
            preferred_element_type=jnp.float32) * scale
        e = jnp.exp(scores + neg[:, None, :])
        attn = (e / jnp.sum(e, axis=-1, keepdims=True)).astype(bf)
        o_h = jax.lax.dot_general(
            attn, v_h.astype(bf), (((2,), (1,)), ((0,), (0,))),
            preferred_element_type=jnp.float32)
        wo_h = wo[hd * dh:(hd + 1) * dh, :].astype(bf)
        o_acc = o_acc + jnp.dot(o_h.reshape(bb * sp, dh).astype(bf), wo_h,
                                preferred_element_type=jnp.float32)
    o_acc = o_acc + bo[...]

    def ln(t, g, b):
        mu = jnp.mean(t, axis=-1, keepdims=True)
        var = jnp.mean((t - mu) ** 2, axis=-1, keepdims=True)
        return (t - mu) / jnp.sqrt(var + 1e-5) * g[...] + b[...]

    x2 = ln(x2 + o_acc, ln1g, ln1b)
    f = jnp.maximum(jnp.dot(x2.astype(bf), w1[...].astype(bf),
                            preferred_element_type=jnp.float32)
                    + b1[...], 0.0)
    f = jnp.dot(f.astype(bf), w2[...].astype(bf),
                preferred_element_type=jnp.float32) + b2[...]
    x2 = ln(x2 + f, ln2g, ln2b)

    enc = x2.reshape(bb, sp, h)
    keep = (1.0 - padf)
    summed = jnp.sum(enc * keep[:, :, None], axis=1)
    cnt = jnp.sum(keep, axis=1, keepdims=True)
    pooled_ref[...] = summed / cnt


def _encoder_pool(emb_all, padf, p, bb):
    twob, sp, h = emb_all.shape
    b = twob // 2
    dh = h // NH
    nblk = twob // bb
    bpb = b // bb
    bs_x = pl.BlockSpec((bb, sp, h), lambda i: (i, 0, 0))
    bs_m = pl.BlockSpec((bb, sp), lambda i: (i % bpb, 0))
    full = lambda *shape: pl.BlockSpec(shape, lambda i: (0,) * len(shape))
    w = lambda a: full(*a.shape)
    body = functools.partial(_encoder_body, bb=bb, dh=dh)
    return pl.pallas_call(
        body,
        grid=(nblk,),
        in_specs=[bs_x, bs_m,
                  w(p['Wq']), full(1, h), w(p['Wk']), full(1, h),
                  w(p['Wv']), full(1, h), w(p['Wo']), full(1, h),
                  full(1, h), full(1, h), full(1, h), full(1, h),
                  w(p['W1']), full(1, p['W1'].shape[1]),
                  w(p['W2']), full(1, h)],
        out_specs=pl.BlockSpec((bb, h), lambda i: (i, 0)),
        out_shape=jax.ShapeDtypeStruct((twob, h), jnp.float32),
    )(emb_all, padf,
      p['Wq'], p['bq'][None], p['Wk'], p['bk'][None],
      p['Wv'], p['bv'][None], p['Wo'], p['bo'][None],
      p['ln1_g'][None], p['ln1_b'][None], p['ln2_g'][None], p['ln2_b'][None],
      p['W1'], p['b1'][None], p['W2'], p['b2'][None])


def kernel(sequences, edge_index1, edge_index2, node_table, Wg1, bg1, Wg2, bg2,
           Wq, bq, Wk, bk, Wv, bv, Wo, bo, ln1_g, ln1_b, ln2_g, ln2_b,
           W1, b1, W2, b2):
    V, D = node_table.shape
    B, S = sequences.shape
    H = Wg1.shape[1]
    E = edge_index1.shape[1]

    grp = 16 * CH * 8
    E_pad = ((E + grp - 1) // grp) * grp
    VP = ((V + 255) // 256) * 256
    npad = E_pad - E
    pad_src = (jnp.arange(npad, dtype=jnp.int32) % V)
    pad_dst = V + (jnp.arange(npad, dtype=jnp.int32) % (VP - V))
    pad_e = jnp.stack([pad_src, pad_dst])
    e1 = jnp.concatenate([edge_index1, pad_e], axis=1) \
        .reshape(2, E_pad // CH, CH).transpose(1, 0, 2)
    e2 = jnp.concatenate([edge_index2, pad_e], axis=1) \
        .reshape(2, E_pad // CH, CH).transpose(1, 0, 2)
    table_q = node_table.reshape(V, NQ, QW).transpose(1, 0, 2)

    seg_k = _sc_segsum(table_q, e1, e2, VP)
    agg1q, agg2q, deg1, deg2 = seg_k(table_q, e1, e2)

    vb = 2000 if V % 2000 == 0 else V
    node_enc1, node_enc2 = _gcn_finish_q(agg1q, deg1[:V], agg2q, deg2[:V],
                                         Wg1, bg1, Wg2, bg2, V, vb)

    sp = S_PAD if S <= S_PAD else S
    seq_pad = jnp.full((B, sp), V, jnp.int32).at[:, :S].set(sequences)
    padf = (seq_pad == V).astype(jnp.float32)
    flat = seq_pad.reshape(-1)
    fill = jnp.arange(flat.shape[0], dtype=jnp.int32) % V
    idx_eff = jnp.where(flat == V, fill, flat)
    idx2d = idx_eff.reshape(-1, 128)
    emb_flat = _sc_lookup(node_enc1, node_enc2, idx2d)
    emb_all = emb_flat.reshape(2 * B, sp, H)

    p = dict(Wq=Wq, bq=bq, Wk=Wk, bk=bk, Wv=Wv, bv=bv, Wo=Wo, bo=bo,
             ln1_g=ln1_g, ln1_b=ln1_b, ln2_g=ln2_g, ln2_b=ln2_b,
             W1=W1, b1=b1, W2=W2, b2=b2)
    bb = 64 if B % 64 == 0 else B
    pooled_all = _encoder_pool(emb_all, padf, p, bb)
    pooled1, pooled2 = pooled_all[:B], pooled_all[B:]
    return (node_enc1, node_enc2, pooled1, pooled2)
